# Initial kernel scaffold; baseline (speedup 1.0000x reference)
import jax
import jax.numpy as jnp
from jax.experimental import pallas as pl


def kernel(x, ei0_cites, ei0_writes, ei1_cites, ei1_writes, W0_cites, al0_cites, ar0_cites, W0_writes, al0_writes, ar0_writes, W1_cites, al1_cites, ar1_cites, W1_writes, al1_writes, ar1_writes, W_lin, b_lin):
    raise NotImplementedError("write your pallas kernel here")


# speedup vs baseline: 49.4801x; 49.4801x over previous
import dataclasses
import functools

import jax
import jax.numpy as jnp
from jax import lax
from jax.experimental import pallas as pl
from jax.experimental.pallas import tpu as pltpu
from jax.experimental.pallas import tpu_sc as plsc

N = 10000
E = 160000
DIM = 128
HEADS = 4
D_HEAD = 32
NPAD = 10240
B = 80
NB = (E // 16) // B
ROWS_PT = NPAD // 16
DEN_R = 512
DEN_PT = DEN_R // 16
BLK = 1000
EPS = 1e-9


def _proj_body(x_ref, w_ref, wlr_ref, z_ref, elr_ref):
    xb = x_ref[...]
    z_ref[0] = jnp.dot(xb, w_ref[0], preferred_element_type=jnp.float32)
    elr_ref[0] = jnp.dot(xb, wlr_ref[0], preferred_element_type=jnp.float32)


def _proj(x, w_stack, wlr_stack):
    return pl.pallas_call(
        _proj_body,
        grid=(2, N // BLK),
        in_specs=[
            pl.BlockSpec((BLK, DIM), lambda e, i: (i, 0)),
            pl.BlockSpec((1, DIM, DIM), lambda e, i: (e, 0, 0)),
            pl.BlockSpec((1, DIM, 8), lambda e, i: (e, 0, 0)),
        ],
        out_specs=[
            pl.BlockSpec((1, BLK, DIM), lambda e, i: (e, i, 0)),
            pl.BlockSpec((1, BLK, 8), lambda e, i: (e, i, 0)),
        ],
        out_shape=[
            jax.ShapeDtypeStruct((2, N, DIM), jnp.float32),
            jax.ShapeDtypeStruct((2, N, 8), jnp.float32),
        ],
    )(x, w_stack, wlr_stack)


def _norm(m_ref, d_ref):
    def one(a, d4):
        dexp = jnp.broadcast_to(d4[:, :, None], (BLK, HEADS, D_HEAD))
        dexp = dexp.reshape(BLK, DIM)
        return a / (dexp + EPS)

    return one(m_ref[0], d_ref[0]) + one(m_ref[1], d_ref[1])


def _mid_body(m_ref, d_ref, w_ref, wlr_ref, z_ref, elr_ref):
    h = jnp.maximum(_norm(m_ref, d_ref), 0.0)
    z_ref[0] = jnp.dot(h, w_ref[0], preferred_element_type=jnp.float32)
    elr_ref[0] = jnp.dot(h, wlr_ref[0], preferred_element_type=jnp.float32)


def _mid(msg, den, w_stack, wlr_stack):
    return pl.pallas_call(
        _mid_body,
        grid=(2, N // BLK),
        in_specs=[
            pl.BlockSpec((2, BLK, DIM), lambda e, i: (0, i, 0)),
            pl.BlockSpec((2, BLK, HEADS), lambda e, i: (0, i, 0)),
            pl.BlockSpec((1, DIM, DIM), lambda e, i: (e, 0, 0)),
            pl.BlockSpec((1, DIM, 8), lambda e, i: (e, 0, 0)),
        ],
        out_specs=[
            pl.BlockSpec((1, BLK, DIM), lambda e, i: (e, i, 0)),
            pl.BlockSpec((1, BLK, 8), lambda e, i: (e, i, 0)),
        ],
        out_shape=[
            jax.ShapeDtypeStruct((2, N, DIM), jnp.float32),
            jax.ShapeDtypeStruct((2, N, 8), jnp.float32),
        ],
    )(msg, den, w_stack, wlr_stack)


def _fin_body(m_ref, d_ref, w_ref, b_ref, y_ref):
    h = _norm(m_ref, d_ref)
    y_ref[...] = jnp.dot(h, w_ref[...], preferred_element_type=jnp.float32) + b_ref[...]


def _fin(msg, den, w_lin, b_lin):
    return pl.pallas_call(
        _fin_body,
        grid=(N // BLK,),
        in_specs=[
            pl.BlockSpec((2, BLK, DIM), lambda i: (0, i, 0)),
            pl.BlockSpec((2, BLK, HEADS), lambda i: (0, i, 0)),
            pl.BlockSpec((DIM, DIM), lambda i: (0, 0)),
            pl.BlockSpec((DIM,), lambda i: (0,)),
        ],
        out_specs=pl.BlockSpec((BLK, DIM), lambda i: (i, 0)),
        out_shape=jax.ShapeDtypeStruct((N, DIM), jnp.float32),
    )(msg, den, w_lin, b_lin)


BA = 400
NBA = (E // 16) // BA


def _sc_logits_body(elr_hbm, srcadj_hbm, dstadj_hbm, w_hbm,
                    elr_t, src_v, dst_v, wbuf):
    cid = lax.axis_index("c")
    sid = lax.axis_index("s")

    pltpu.sync_copy(elr_hbm.at[cid], elr_t)

    ebase = cid * E + sid * (E // 16)
    nloc8 = cid * (N * 8)

    @pl.loop(0, NBA)
    def _batch(b):
        base = ebase + b * BA
        pltpu.sync_copy(srcadj_hbm.at[pl.ds(base, BA)], src_v)
        pltpu.sync_copy(dstadj_hbm.at[pl.ds(base, BA)], dst_v)

        for g in range(BA // 16):
            sv = src_v[pl.ds(g * 16, 16)] * 8 - nloc8
            dv = dst_v[pl.ds(g * 16, 16)] * 8 - nloc8
            rows = lax.iota(jnp.int32, 16) + g * 16
            for h in range(HEADS):
                elv = plsc.load_gather(elr_t, [sv + h])
                erv = plsc.load_gather(elr_t, [dv + (4 + h)])
                e = elv + erv
                e = jnp.where(e >= 0.0, e, e * jnp.float32(0.2))
                plsc.store_scatter(wbuf, [rows * 4 + h], jnp.exp(e))

        pltpu.sync_copy(wbuf, w_hbm.at[pl.ds(base * 4, BA * 4)])


def _sc_agg_body(z_hbm, w_hbm, srcadj_hbm, dstadj_hbm, msg_hbm, den_hbm,
                 src_v, dst_v, dstloc_v, dstden_v, zbuf, uden, wbuf, acc, den,
                 sem):
    cid = lax.axis_index("c")
    sid = lax.axis_index("s")

    @pl.loop(0, B)
    def _zero_bufs(r):
        for c in range(DIM // 16):
            zbuf[r, pl.ds(c * 16, 16)] = jnp.zeros((16,), jnp.float32)
            uden[r, pl.ds(c * 16, 16)] = jnp.zeros((16,), jnp.float32)

    r0 = sid * ROWS_PT
    d0 = sid * DEN_PT

    @pl.loop(0, ROWS_PT // B)
    def _zero_acc(j):
        pltpu.sync_copy(zbuf, acc.at[pl.ds(r0 + j * B, B)])

    pltpu.sync_copy(zbuf.at[pl.ds(0, DEN_PT)], den.at[pl.ds(d0, DEN_PT)])
    plsc.subcore_barrier()

    ebase = cid * E + sid * (E // 16)
    nloc = cid * N

    @pl.loop(0, NB)
    def _batch(b):
        base = ebase + b * B
        pltpu.sync_copy(srcadj_hbm.at[pl.ds(base, B)], src_v)
        pltpu.sync_copy(dstadj_hbm.at[pl.ds(base, B)], dst_v)
        gat = pltpu.async_copy(z_hbm.at[src_v], zbuf, sem)
        pltpu.sync_copy(w_hbm.at[pl.ds(base * 4, B * 4)], wbuf)

        for g in range(B // 16):
            dv = dst_v[pl.ds(g * 16, 16)] - nloc
            dstloc_v[pl.ds(g * 16, 16)] = dv
            dstden_v[pl.ds(g * 16, 16)] = lax.shift_right_logical(dv, 5)
            dcol = (dv & 31) * 4
            rows = lax.iota(jnp.int32, 16) + g * 16
            for h in range(HEADS):
                w = plsc.load_gather(wbuf, [rows * 4 + h])
                plsc.store_scatter(uden, [rows, dcol + h], w)

        gat.wait()

        for g in range(B // 16):
            wv = [plsc.load_gather(wbuf, [(lax.iota(jnp.int32, 16) + g * 16) * 4 + h])
                  for h in range(HEADS)]
            for j in range(16):
                r = g * 16 + j
                for h in range(HEADS):
                    ws = wv[h][j]
                    for k in range(2):
                        off = h * D_HEAD + k * 16
                        zbuf[r, pl.ds(off, 16)] = zbuf[r, pl.ds(off, 16)] * ws

        pltpu.sync_copy(zbuf, acc.at[dstloc_v], add=True)
        pltpu.sync_copy(uden, den.at[dstden_v], add=True)

        for g in range(B // 16):
            dv = dstloc_v[pl.ds(g * 16, 16)]
            dcol = (dv & 31) * 4
            rows = lax.iota(jnp.int32, 16) + g * 16
            z16 = jnp.zeros((16,), jnp.float32)
            for h in range(HEADS):
                plsc.store_scatter(uden, [rows, dcol + h], z16)

    plsc.subcore_barrier()

    @pl.loop(0, ROWS_PT // B)
    def _flush(j):
        pltpu.sync_copy(acc.at[pl.ds(r0 + j * B, B)], zbuf)
        pltpu.sync_copy(zbuf, msg_hbm.at[cid, pl.ds(r0 + j * B, B)])

    pltpu.sync_copy(den.at[pl.ds(d0, DEN_PT)], zbuf.at[pl.ds(0, DEN_PT)])
    pltpu.sync_copy(zbuf.at[pl.ds(0, DEN_PT)], den_hbm.at[cid, pl.ds(d0, DEN_PT)])


def _sc_params():
    cp = pltpu.CompilerParams()
    if "needs_layout_passes" in pltpu.CompilerParams.__dataclass_fields__:
        cp = dataclasses.replace(cp, needs_layout_passes=False)
    return cp


@jax.jit
def _sc_layer(z_flat, elr_flat, srcadj, dstadj):
    mesh = plsc.VectorSubcoreMesh(core_axis_name="c", subcore_axis_name="s")
    logits = functools.partial(
        pl.kernel,
        mesh=mesh,
        compiler_params=_sc_params(),
        out_type=jax.ShapeDtypeStruct((2 * E * 4,), jnp.float32),
        scratch_types=[
            pltpu.VMEM((N * 8,), jnp.float32),
            pltpu.VMEM((BA,), jnp.int32),
            pltpu.VMEM((BA,), jnp.int32),
            pltpu.VMEM((BA * 4,), jnp.float32),
        ],
    )(_sc_logits_body)
    w_edges = logits(elr_flat, srcadj, dstadj)

    agg = functools.partial(
        pl.kernel,
        mesh=mesh,
        compiler_params=_sc_params(),
        out_type=[
            jax.ShapeDtypeStruct((2, NPAD, DIM), jnp.float32),
            jax.ShapeDtypeStruct((2, DEN_R, DIM), jnp.float32),
        ],
        scratch_types=[
            pltpu.VMEM((B,), jnp.int32),
            pltpu.VMEM((B,), jnp.int32),
            pltpu.VMEM((B,), jnp.int32),
            pltpu.VMEM((B,), jnp.int32),
            pltpu.VMEM((B, DIM), jnp.float32),
            pltpu.VMEM((B, DIM), jnp.float32),
            pltpu.VMEM((B * 4,), jnp.float32),
            pltpu.VMEM_SHARED((NPAD, DIM), jnp.float32),
            pltpu.VMEM_SHARED((DEN_R, DIM), jnp.float32),
            pltpu.SemaphoreType.DMA,
        ],
    )(_sc_agg_body)
    return agg(z_flat, w_edges, srcadj, dstadj)


def _fold(W, a):
    return jnp.sum(W.reshape(DIM, HEADS, D_HEAD) * a[None, :, :], axis=-1)


def _wlr(W, al, ar):
    return jnp.concatenate([_fold(W, al), _fold(W, ar)], axis=1)


def kernel(x, ei0_cites, ei0_writes, ei1_cites, ei1_writes, W0_cites, al0_cites, ar0_cites, W0_writes, al0_writes, ar0_writes, W1_cites, al1_cites, ar1_cites, W1_writes, al1_writes, ar1_writes, W_lin, b_lin):
    w0 = jnp.stack([W0_cites, W0_writes])
    wlr0 = jnp.stack([_wlr(W0_cites, al0_cites, ar0_cites),
                      _wlr(W0_writes, al0_writes, ar0_writes)])
    w1 = jnp.stack([W1_cites, W1_writes])
    wlr1 = jnp.stack([_wlr(W1_cites, al1_cites, ar1_cites),
                      _wlr(W1_writes, al1_writes, ar1_writes)])

    srcadj0 = jnp.concatenate([ei0_cites[0], ei0_writes[0] + N])
    dstadj0 = jnp.concatenate([ei0_cites[1], ei0_writes[1] + N])
    srcadj1 = jnp.concatenate([ei1_cites[0], ei1_writes[0] + N])
    dstadj1 = jnp.concatenate([ei1_cites[1], ei1_writes[1] + N])

    z0, elr0 = _proj(x, w0, wlr0)
    msg0, den0 = _sc_layer(z0.reshape(2 * N, DIM), elr0.reshape(2, N * 8),
                           srcadj0, dstadj0)
    z1, elr1 = _mid(msg0, den0.reshape(2, DEN_R * 32, HEADS), w1, wlr1)
    msg1, den1 = _sc_layer(z1.reshape(2 * N, DIM), elr1.reshape(2, N * 8),
                           srcadj1, dstadj1)
    return _fin(msg1, den1.reshape(2, DEN_R * 32, HEADS), W_lin, b_lin)


# speedup vs baseline: 73.9635x; 1.4948x over previous
import dataclasses
import functools

import jax
import jax.numpy as jnp
from jax import lax
from jax.experimental import pallas as pl
from jax.experimental.pallas import tpu as pltpu
from jax.experimental.pallas import tpu_sc as plsc

N = 10000
E = 160000
DIM = 128
HEADS = 4
D_HEAD = 32
NPAD = 10240
B = 80
NB = (E // 16) // B
ROWS_PT = NPAD // 16
DEN_R = 512
DEN_PT = DEN_R // 16
BLK = 1000
EPS = 1e-9


def _proj_body(x_ref, w_ref, wlr_ref, z_ref, elr_ref):
    xb = x_ref[...]
    z_ref[0] = jnp.dot(xb, w_ref[0], preferred_element_type=jnp.float32)
    elr_ref[0] = jnp.dot(xb, wlr_ref[0], preferred_element_type=jnp.float32)


def _proj(x, w_stack, wlr_stack):
    return pl.pallas_call(
        _proj_body,
        grid=(2, N // BLK),
        in_specs=[
            pl.BlockSpec((BLK, DIM), lambda e, i: (i, 0)),
            pl.BlockSpec((1, DIM, DIM), lambda e, i: (e, 0, 0)),
            pl.BlockSpec((1, DIM, 8), lambda e, i: (e, 0, 0)),
        ],
        out_specs=[
            pl.BlockSpec((1, BLK, DIM), lambda e, i: (e, i, 0)),
            pl.BlockSpec((1, BLK, 8), lambda e, i: (e, i, 0)),
        ],
        out_shape=[
            jax.ShapeDtypeStruct((2, N, DIM), jnp.float32),
            jax.ShapeDtypeStruct((2, N, 8), jnp.float32),
        ],
    )(x, w_stack, wlr_stack)


def _norm(m_ref, d_ref):
    def one(a, d4):
        dexp = jnp.broadcast_to(d4[:, :, None], (BLK, HEADS, D_HEAD))
        dexp = dexp.reshape(BLK, DIM)
        return a / (dexp + EPS)

    return one(m_ref[0], d_ref[0]) + one(m_ref[1], d_ref[1])


def _mid_body(m_ref, d_ref, w_ref, wlr_ref, z_ref, elr_ref):
    h = jnp.maximum(_norm(m_ref, d_ref), 0.0)
    z_ref[0] = jnp.dot(h, w_ref[0], preferred_element_type=jnp.float32)
    elr_ref[0] = jnp.dot(h, wlr_ref[0], preferred_element_type=jnp.float32)


def _mid(msg, den, w_stack, wlr_stack):
    return pl.pallas_call(
        _mid_body,
        grid=(2, N // BLK),
        in_specs=[
            pl.BlockSpec((2, BLK, DIM), lambda e, i: (0, i, 0)),
            pl.BlockSpec((2, BLK, HEADS), lambda e, i: (0, i, 0)),
            pl.BlockSpec((1, DIM, DIM), lambda e, i: (e, 0, 0)),
            pl.BlockSpec((1, DIM, 8), lambda e, i: (e, 0, 0)),
        ],
        out_specs=[
            pl.BlockSpec((1, BLK, DIM), lambda e, i: (e, i, 0)),
            pl.BlockSpec((1, BLK, 8), lambda e, i: (e, i, 0)),
        ],
        out_shape=[
            jax.ShapeDtypeStruct((2, N, DIM), jnp.float32),
            jax.ShapeDtypeStruct((2, N, 8), jnp.float32),
        ],
    )(msg, den, w_stack, wlr_stack)


def _fin_body(m_ref, d_ref, w_ref, b_ref, y_ref):
    h = _norm(m_ref, d_ref)
    y_ref[...] = jnp.dot(h, w_ref[...], preferred_element_type=jnp.float32) + b_ref[...]


def _fin(msg, den, w_lin, b_lin):
    return pl.pallas_call(
        _fin_body,
        grid=(N // BLK,),
        in_specs=[
            pl.BlockSpec((2, BLK, DIM), lambda i: (0, i, 0)),
            pl.BlockSpec((2, BLK, HEADS), lambda i: (0, i, 0)),
            pl.BlockSpec((DIM, DIM), lambda i: (0, 0)),
            pl.BlockSpec((DIM,), lambda i: (0,)),
        ],
        out_specs=pl.BlockSpec((BLK, DIM), lambda i: (i, 0)),
        out_shape=jax.ShapeDtypeStruct((N, DIM), jnp.float32),
    )(msg, den, w_lin, b_lin)


BA = 400
NBA = (E // 16) // BA


def _sc_logits_body(elr_hbm, srcadj_hbm, dstadj_hbm, w_hbm,
                    elr_t, src_v, dst_v, wbuf):
    cid = lax.axis_index("c")
    sid = lax.axis_index("s")

    pltpu.sync_copy(elr_hbm.at[cid], elr_t)

    ebase = cid * E + sid * (E // 16)
    nloc8 = cid * (N * 8)

    @pl.loop(0, NBA)
    def _batch(b):
        base = ebase + b * BA
        pltpu.sync_copy(srcadj_hbm.at[pl.ds(base, BA)], src_v)
        pltpu.sync_copy(dstadj_hbm.at[pl.ds(base, BA)], dst_v)

        for g in range(BA // 16):
            sv = src_v[pl.ds(g * 16, 16)] * 8 - nloc8
            dv = dst_v[pl.ds(g * 16, 16)] * 8 - nloc8
            rows = lax.iota(jnp.int32, 16) + g * 16
            for h in range(HEADS):
                elv = plsc.load_gather(elr_t, [sv + h])
                erv = plsc.load_gather(elr_t, [dv + (4 + h)])
                e = elv + erv
                e = jnp.where(e >= 0.0, e, e * jnp.float32(0.2))
                plsc.store_scatter(wbuf, [rows * 4 + h], jnp.exp(e))

        pltpu.sync_copy(wbuf, w_hbm.at[pl.ds(base * 4, BA * 4)])


def _sc_agg_body(z_hbm, w_hbm, srcadj_hbm, dstadj_hbm, msg_hbm, den_hbm,
                 src0, src1, dst0, dst1, dl0, dl1, dd0, dd1, zb0, zb1,
                 ud0, ud1, wb0, wb1, acc, den,
                 ssrc0, ssrc1, sdst0, sdst1, sz0, sz1, sw0, sw1,
                 sa0, sa1, sd0, sd1):
    cid = lax.axis_index("c")
    sid = lax.axis_index("s")

    SRC = (src0, src1)
    DST = (dst0, dst1)
    DL = (dl0, dl1)
    DD = (dd0, dd1)
    ZB = (zb0, zb1)
    UD = (ud0, ud1)
    WB = (wb0, wb1)
    SSRC = (ssrc0, ssrc1)
    SDST = (sdst0, sdst1)
    SZ = (sz0, sz1)
    SW = (sw0, sw1)
    SA = (sa0, sa1)
    SD = (sd0, sd1)

    @pl.loop(0, B)
    def _zero_bufs(r):
        for c in range(DIM // 16):
            zb0[r, pl.ds(c * 16, 16)] = jnp.zeros((16,), jnp.float32)
            ud0[r, pl.ds(c * 16, 16)] = jnp.zeros((16,), jnp.float32)
            ud1[r, pl.ds(c * 16, 16)] = jnp.zeros((16,), jnp.float32)

    for g in range(B // 16):
        dl0[pl.ds(g * 16, 16)] = jnp.zeros((16,), jnp.int32)
        dl1[pl.ds(g * 16, 16)] = jnp.zeros((16,), jnp.int32)

    r0 = sid * ROWS_PT
    d0 = sid * DEN_PT

    @pl.loop(0, ROWS_PT // B)
    def _zero_acc(j):
        pltpu.sync_copy(zb0, acc.at[pl.ds(r0 + j * B, B)])

    pltpu.sync_copy(zb0.at[pl.ds(0, DEN_PT)], den.at[pl.ds(d0, DEN_PT)])
    plsc.subcore_barrier()

    ebase = cid * E + sid * (E // 16)
    nloc = cid * N

    def issue_idx(bi, p):
        base = ebase + bi * B
        pltpu.async_copy(srcadj_hbm.at[pl.ds(base, B)], SRC[p], SSRC[p])
        pltpu.async_copy(dstadj_hbm.at[pl.ds(base, B)], DST[p], SDST[p])

    def wait_idx(p):
        pltpu.make_async_copy(srcadj_hbm.at[pl.ds(0, B)], SRC[p], SSRC[p]).wait()
        pltpu.make_async_copy(dstadj_hbm.at[pl.ds(0, B)], DST[p], SDST[p]).wait()

    def issue_zw(bi, p):
        base = ebase + bi * B
        pltpu.async_copy(z_hbm.at[SRC[p]], ZB[p], SZ[p])
        pltpu.async_copy(w_hbm.at[pl.ds(base * 4, B * 4)], WB[p], SW[p])

    def wait_sa(p):
        pltpu.make_async_copy(ZB[p], acc.at[DL[p]], SA[p]).wait()

    def wait_sd(p):
        pltpu.make_async_copy(UD[p], den.at[DD[p]], SD[p]).wait()

    def phase(bi, p):
        @pl.when(bi >= 2)
        def _(): wait_sd(p)

        for g in range(B // 16):
            dv = DL[p][pl.ds(g * 16, 16)]
            dcol = (dv & 31) * 4
            rows = lax.iota(jnp.int32, 16) + g * 16
            z16 = jnp.zeros((16,), jnp.float32)
            for h in range(HEADS):
                plsc.store_scatter(UD[p], [rows, dcol + h], z16)

        pltpu.make_async_copy(w_hbm.at[pl.ds(0, B * 4)], WB[p], SW[p]).wait()
        for g in range(B // 16):
            dv = DST[p][pl.ds(g * 16, 16)] - nloc
            DL[p][pl.ds(g * 16, 16)] = dv
            DD[p][pl.ds(g * 16, 16)] = lax.shift_right_logical(dv, 5)
            dcol = (dv & 31) * 4
            rows = lax.iota(jnp.int32, 16) + g * 16
            for h in range(HEADS):
                w = plsc.load_gather(WB[p], [rows * 4 + h])
                plsc.store_scatter(UD[p], [rows, dcol + h], w)

        pltpu.make_async_copy(z_hbm.at[SRC[p]], ZB[p], SZ[p]).wait()

        @pl.loop(0, B // 16)
        def _scale(g):
            wv = [plsc.load_gather(WB[p], [(lax.iota(jnp.int32, 16) + g * 16) * 4 + h])
                  for h in range(HEADS)]
            for j in range(16):
                for h in range(HEADS):
                    ws = wv[h][j]
                    for k in range(2):
                        off = h * D_HEAD + k * 16
                        zrow = g * 16 + j
                        ZB[p][zrow, pl.ds(off, 16)] = ZB[p][zrow, pl.ds(off, 16)] * ws

        pltpu.async_copy(ZB[p], acc.at[DL[p]], SA[p], add=True)
        pltpu.async_copy(UD[p], den.at[DD[p]], SD[p], add=True)

        q = 1 - p

        @pl.when(bi + 1 < NB)
        def _():
            wait_idx(q)

            @pl.when(bi + 1 >= 2)
            def _(): wait_sa(q)

            issue_zw(bi + 1, q)

            @pl.when(bi + 2 < NB)
            def _(): issue_idx(bi + 2, p)

    issue_idx(0, 0)
    wait_idx(0)
    issue_zw(0, 0)
    issue_idx(1, 1)

    @pl.loop(0, NB, step=2)
    def _batch(bi):
        phase(bi, 0)

        @pl.when(bi + 1 < NB)
        def _(): phase(bi + 1, 1)

    wait_sa(0)
    wait_sd(0)
    wait_sa(1)
    wait_sd(1)
    plsc.subcore_barrier()

    @pl.loop(0, ROWS_PT // B)
    def _flush(j):
        pltpu.sync_copy(acc.at[pl.ds(r0 + j * B, B)], zb0)
        pltpu.sync_copy(zb0, msg_hbm.at[cid, pl.ds(r0 + j * B, B)])

    pltpu.sync_copy(den.at[pl.ds(d0, DEN_PT)], zb0.at[pl.ds(0, DEN_PT)])
    pltpu.sync_copy(zb0.at[pl.ds(0, DEN_PT)], den_hbm.at[cid, pl.ds(d0, DEN_PT)])


def _sc_params():
    cp = pltpu.CompilerParams()
    if "needs_layout_passes" in pltpu.CompilerParams.__dataclass_fields__:
        cp = dataclasses.replace(cp, needs_layout_passes=False)
    return cp


@jax.jit
def _sc_layer(z_flat, elr_flat, srcadj, dstadj):
    mesh = plsc.VectorSubcoreMesh(core_axis_name="c", subcore_axis_name="s")
    logits = functools.partial(
        pl.kernel,
        mesh=mesh,
        compiler_params=_sc_params(),
        out_type=jax.ShapeDtypeStruct((2 * E * 4,), jnp.float32),
        scratch_types=[
            pltpu.VMEM((N * 8,), jnp.float32),
            pltpu.VMEM((BA,), jnp.int32),
            pltpu.VMEM((BA,), jnp.int32),
            pltpu.VMEM((BA * 4,), jnp.float32),
        ],
    )(_sc_logits_body)
    w_edges = logits(elr_flat, srcadj, dstadj)

    agg = functools.partial(
        pl.kernel,
        mesh=mesh,
        compiler_params=_sc_params(),
        out_type=[
            jax.ShapeDtypeStruct((2, NPAD, DIM), jnp.float32),
            jax.ShapeDtypeStruct((2, DEN_R, DIM), jnp.float32),
        ],
        scratch_types=(
            [pltpu.VMEM((B,), jnp.int32)] * 8 +
            [pltpu.VMEM((B, DIM), jnp.float32)] * 2 +
            [pltpu.VMEM((B, DIM), jnp.float32)] * 2 +
            [pltpu.VMEM((B * 4,), jnp.float32)] * 2 +
            [pltpu.VMEM_SHARED((NPAD, DIM), jnp.float32),
             pltpu.VMEM_SHARED((DEN_R, DIM), jnp.float32)] +
            [pltpu.SemaphoreType.DMA] * 12
        ),
    )(_sc_agg_body)
    return agg(z_flat, w_edges, srcadj, dstadj)


def _fold(W, a):
    return jnp.sum(W.reshape(DIM, HEADS, D_HEAD) * a[None, :, :], axis=-1)


def _wlr(W, al, ar):
    return jnp.concatenate([_fold(W, al), _fold(W, ar)], axis=1)


def kernel(x, ei0_cites, ei0_writes, ei1_cites, ei1_writes, W0_cites, al0_cites, ar0_cites, W0_writes, al0_writes, ar0_writes, W1_cites, al1_cites, ar1_cites, W1_writes, al1_writes, ar1_writes, W_lin, b_lin):
    w0 = jnp.stack([W0_cites, W0_writes])
    wlr0 = jnp.stack([_wlr(W0_cites, al0_cites, ar0_cites),
                      _wlr(W0_writes, al0_writes, ar0_writes)])
    w1 = jnp.stack([W1_cites, W1_writes])
    wlr1 = jnp.stack([_wlr(W1_cites, al1_cites, ar1_cites),
                      _wlr(W1_writes, al1_writes, ar1_writes)])

    srcadj0 = jnp.concatenate([ei0_cites[0], ei0_writes[0] + N])
    dstadj0 = jnp.concatenate([ei0_cites[1], ei0_writes[1] + N])
    srcadj1 = jnp.concatenate([ei1_cites[0], ei1_writes[0] + N])
    dstadj1 = jnp.concatenate([ei1_cites[1], ei1_writes[1] + N])

    z0, elr0 = _proj(x, w0, wlr0)
    msg0, den0 = _sc_layer(z0.reshape(2 * N, DIM), elr0.reshape(2, N * 8),
                           srcadj0, dstadj0)
    z1, elr1 = _mid(msg0, den0.reshape(2, DEN_R * 32, HEADS), w1, wlr1)
    msg1, den1 = _sc_layer(z1.reshape(2 * N, DIM), elr1.reshape(2, N * 8),
                           srcadj1, dstadj1)
    return _fin(msg1, den1.reshape(2, DEN_R * 32, HEADS), W_lin, b_lin)


# speedup vs baseline: 79.4452x; 1.0741x over previous
import dataclasses
import functools

import jax
import jax.numpy as jnp
from jax import lax
from jax.experimental import pallas as pl
from jax.experimental.pallas import tpu as pltpu
from jax.experimental.pallas import tpu_sc as plsc

N = 10000
E = 160000
DIM = 128
HEADS = 4
D_HEAD = 32
NPAD = 10240
B = 80
NB = (E // 16) // B
ROWS_PT = NPAD // 16
DEN_R = 512
DEN_PT = DEN_R // 16
BLK = 1000
EPS = 1e-9


def _proj_body(x_ref, w_ref, wlr_ref, z_ref, elr_ref):
    xb = x_ref[...]
    z_ref[0] = jnp.dot(xb, w_ref[0], preferred_element_type=jnp.float32)
    elr_ref[0] = jnp.dot(xb, wlr_ref[0], preferred_element_type=jnp.float32)


def _proj(x, w_stack, wlr_stack):
    return pl.pallas_call(
        _proj_body,
        grid=(2, N // BLK),
        in_specs=[
            pl.BlockSpec((BLK, DIM), lambda e, i: (i, 0)),
            pl.BlockSpec((1, DIM, DIM), lambda e, i: (e, 0, 0)),
            pl.BlockSpec((1, DIM, 8), lambda e, i: (e, 0, 0)),
        ],
        out_specs=[
            pl.BlockSpec((1, BLK, DIM), lambda e, i: (e, i, 0)),
            pl.BlockSpec((1, BLK, 8), lambda e, i: (e, i, 0)),
        ],
        out_shape=[
            jax.ShapeDtypeStruct((2, N, DIM), jnp.float32),
            jax.ShapeDtypeStruct((2, N, 8), jnp.float32),
        ],
    )(x, w_stack, wlr_stack)


def _norm(m_ref, d_ref):
    def one(a, d4):
        dexp = jnp.broadcast_to(d4[:, :, None], (BLK, HEADS, D_HEAD))
        dexp = dexp.reshape(BLK, DIM)
        return a / (dexp + EPS)

    return one(m_ref[0], d_ref[0]) + one(m_ref[1], d_ref[1])


def _mid_body(m_ref, d_ref, w_ref, wlr_ref, z_ref, elr_ref):
    h = jnp.maximum(_norm(m_ref, d_ref), 0.0)
    z_ref[0] = jnp.dot(h, w_ref[0], preferred_element_type=jnp.float32)
    elr_ref[0] = jnp.dot(h, wlr_ref[0], preferred_element_type=jnp.float32)


def _mid(msg, den, w_stack, wlr_stack):
    return pl.pallas_call(
        _mid_body,
        grid=(2, N // BLK),
        in_specs=[
            pl.BlockSpec((2, BLK, DIM), lambda e, i: (0, i, 0)),
            pl.BlockSpec((2, BLK, HEADS), lambda e, i: (0, i, 0)),
            pl.BlockSpec((1, DIM, DIM), lambda e, i: (e, 0, 0)),
            pl.BlockSpec((1, DIM, 8), lambda e, i: (e, 0, 0)),
        ],
        out_specs=[
            pl.BlockSpec((1, BLK, DIM), lambda e, i: (e, i, 0)),
            pl.BlockSpec((1, BLK, 8), lambda e, i: (e, i, 0)),
        ],
        out_shape=[
            jax.ShapeDtypeStruct((2, N, DIM), jnp.float32),
            jax.ShapeDtypeStruct((2, N, 8), jnp.float32),
        ],
    )(msg, den, w_stack, wlr_stack)


def _fin_body(m_ref, d_ref, w_ref, b_ref, y_ref):
    h = _norm(m_ref, d_ref)
    y_ref[...] = jnp.dot(h, w_ref[...], preferred_element_type=jnp.float32) + b_ref[...]


def _fin(msg, den, w_lin, b_lin):
    return pl.pallas_call(
        _fin_body,
        grid=(N // BLK,),
        in_specs=[
            pl.BlockSpec((2, BLK, DIM), lambda i: (0, i, 0)),
            pl.BlockSpec((2, BLK, HEADS), lambda i: (0, i, 0)),
            pl.BlockSpec((DIM, DIM), lambda i: (0, 0)),
            pl.BlockSpec((DIM,), lambda i: (0,)),
        ],
        out_specs=pl.BlockSpec((BLK, DIM), lambda i: (i, 0)),
        out_shape=jax.ShapeDtypeStruct((N, DIM), jnp.float32),
    )(msg, den, w_lin, b_lin)


BA = 400
NBA = (E // 16) // BA


def _sc_logits_body(elr_hbm, srcadj_hbm, dstadj_hbm, w_hbm,
                    elr_t, src0, src1, dst0, dst1, wb0, wb1,
                    ssrc0, ssrc1, sdst0, sdst1, sww0, sww1):
    cid = lax.axis_index("c")
    sid = lax.axis_index("s")

    SRC = (src0, src1)
    DST = (dst0, dst1)
    WB = (wb0, wb1)
    SSRC = (ssrc0, ssrc1)
    SDST = (sdst0, sdst1)
    SWW = (sww0, sww1)

    pltpu.sync_copy(elr_hbm.at[cid], elr_t)

    ebase = cid * E + sid * (E // 16)
    nloc8 = cid * (N * 8)

    def issue_idx(bi, p):
        base = ebase + bi * BA
        pltpu.async_copy(srcadj_hbm.at[pl.ds(base, BA)], SRC[p], SSRC[p])
        pltpu.async_copy(dstadj_hbm.at[pl.ds(base, BA)], DST[p], SDST[p])

    def wait_idx(p):
        pltpu.make_async_copy(srcadj_hbm.at[pl.ds(0, BA)], SRC[p], SSRC[p]).wait()
        pltpu.make_async_copy(dstadj_hbm.at[pl.ds(0, BA)], DST[p], SDST[p]).wait()

    def wait_w(p):
        pltpu.make_async_copy(WB[p], w_hbm.at[pl.ds(0, BA * 4)], SWW[p]).wait()

    def phase(bi, p):
        @pl.when(bi >= 2)
        def _(): wait_w(p)

        @pl.loop(0, BA // 16)
        def _grp(g):
            sv = SRC[p][pl.ds(g * 16, 16)] * 8 - nloc8
            dv = DST[p][pl.ds(g * 16, 16)] * 8 - nloc8
            rows = lax.iota(jnp.int32, 16) + g * 16
            for h in range(HEADS):
                elv = plsc.load_gather(elr_t, [sv + h])
                erv = plsc.load_gather(elr_t, [dv + (4 + h)])
                e = elv + erv
                e = jnp.where(e >= 0.0, e, e * jnp.float32(0.2))
                plsc.store_scatter(WB[p], [rows * 4 + h], jnp.exp(e))

        base = ebase + bi * BA
        pltpu.async_copy(WB[p], w_hbm.at[pl.ds(base * 4, BA * 4)], SWW[p])

        q = 1 - p

        @pl.when(bi + 1 < NBA)
        def _():
            wait_idx(q)

            @pl.when(bi + 2 < NBA)
            def _(): issue_idx(bi + 2, p)

    issue_idx(0, 0)
    wait_idx(0)
    issue_idx(1, 1)

    @pl.loop(0, NBA, step=2)
    def _batch(bi):
        phase(bi, 0)

        @pl.when(bi + 1 < NBA)
        def _(): phase(bi + 1, 1)

    wait_w(0)
    wait_w(1)


def _sc_agg_body(z_hbm, w_hbm, srcadj_hbm, dstadj_hbm, msg_hbm, den_hbm,
                 src0, src1, dst0, dst1, dl0, dl1, dd0, dd1, zb0, zb1,
                 ud0, ud1, wb0, wb1, acc, den,
                 ssrc0, ssrc1, sdst0, sdst1, sz0, sz1, sw0, sw1,
                 sa0, sa1, sd0, sd1):
    cid = lax.axis_index("c")
    sid = lax.axis_index("s")

    SRC = (src0, src1)
    DST = (dst0, dst1)
    DL = (dl0, dl1)
    DD = (dd0, dd1)
    ZB = (zb0, zb1)
    UD = (ud0, ud1)
    WB = (wb0, wb1)
    SSRC = (ssrc0, ssrc1)
    SDST = (sdst0, sdst1)
    SZ = (sz0, sz1)
    SW = (sw0, sw1)
    SA = (sa0, sa1)
    SD = (sd0, sd1)

    @pl.loop(0, B)
    def _zero_bufs(r):
        for c in range(DIM // 16):
            zb0[r, pl.ds(c * 16, 16)] = jnp.zeros((16,), jnp.float32)
            ud0[r, pl.ds(c * 16, 16)] = jnp.zeros((16,), jnp.float32)
            ud1[r, pl.ds(c * 16, 16)] = jnp.zeros((16,), jnp.float32)

    for g in range(B // 16):
        dl0[pl.ds(g * 16, 16)] = jnp.zeros((16,), jnp.int32)
        dl1[pl.ds(g * 16, 16)] = jnp.zeros((16,), jnp.int32)

    r0 = sid * ROWS_PT
    d0 = sid * DEN_PT

    @pl.loop(0, ROWS_PT // B)
    def _zero_acc(j):
        pltpu.sync_copy(zb0, acc.at[pl.ds(r0 + j * B, B)])

    pltpu.sync_copy(zb0.at[pl.ds(0, DEN_PT)], den.at[pl.ds(d0, DEN_PT)])
    plsc.subcore_barrier()

    ebase = cid * E + sid * (E // 16)
    nloc = cid * N

    def issue_idx(bi, p):
        base = ebase + bi * B
        pltpu.async_copy(srcadj_hbm.at[pl.ds(base, B)], SRC[p], SSRC[p])
        pltpu.async_copy(dstadj_hbm.at[pl.ds(base, B)], DST[p], SDST[p])

    def wait_idx(p):
        pltpu.make_async_copy(srcadj_hbm.at[pl.ds(0, B)], SRC[p], SSRC[p]).wait()
        pltpu.make_async_copy(dstadj_hbm.at[pl.ds(0, B)], DST[p], SDST[p]).wait()

    def issue_zw(bi, p):
        base = ebase + bi * B
        pltpu.async_copy(z_hbm.at[SRC[p]], ZB[p], SZ[p])
        pltpu.async_copy(w_hbm.at[pl.ds(base * 4, B * 4)], WB[p], SW[p])

    def wait_sa(p):
        pltpu.make_async_copy(ZB[p], acc.at[DL[p]], SA[p]).wait()

    def wait_sd(p):
        pltpu.make_async_copy(UD[p], den.at[DD[p]], SD[p]).wait()

    def phase(bi, p):
        @pl.when(bi >= 2)
        def _(): wait_sd(p)

        for g in range(B // 16):
            dv = DL[p][pl.ds(g * 16, 16)]
            dcol = (dv & 31) * 4
            rows = lax.iota(jnp.int32, 16) + g * 16
            z16 = jnp.zeros((16,), jnp.float32)
            for h in range(HEADS):
                plsc.store_scatter(UD[p], [rows, dcol + h], z16)

        pltpu.make_async_copy(w_hbm.at[pl.ds(0, B * 4)], WB[p], SW[p]).wait()
        for g in range(B // 16):
            dv = DST[p][pl.ds(g * 16, 16)] - nloc
            DL[p][pl.ds(g * 16, 16)] = dv
            DD[p][pl.ds(g * 16, 16)] = lax.shift_right_logical(dv, 5)
            dcol = (dv & 31) * 4
            rows = lax.iota(jnp.int32, 16) + g * 16
            for h in range(HEADS):
                w = plsc.load_gather(WB[p], [rows * 4 + h])
                plsc.store_scatter(UD[p], [rows, dcol + h], w)

        pltpu.make_async_copy(z_hbm.at[SRC[p]], ZB[p], SZ[p]).wait()

        @pl.loop(0, B // 16)
        def _scale(g):
            wv = [plsc.load_gather(WB[p], [(lax.iota(jnp.int32, 16) + g * 16) * 4 + h])
                  for h in range(HEADS)]
            for j in range(16):
                for h in range(HEADS):
                    ws = wv[h][j]
                    for k in range(2):
                        off = h * D_HEAD + k * 16
                        zrow = g * 16 + j
                        ZB[p][zrow, pl.ds(off, 16)] = ZB[p][zrow, pl.ds(off, 16)] * ws

        pltpu.async_copy(ZB[p], acc.at[DL[p]], SA[p], add=True)
        pltpu.async_copy(UD[p], den.at[DD[p]], SD[p], add=True)

        q = 1 - p

        @pl.when(bi + 1 < NB)
        def _():
            wait_idx(q)

            @pl.when(bi + 1 >= 2)
            def _(): wait_sa(q)

            issue_zw(bi + 1, q)

            @pl.when(bi + 2 < NB)
            def _(): issue_idx(bi + 2, p)

    issue_idx(0, 0)
    wait_idx(0)
    issue_zw(0, 0)
    issue_idx(1, 1)

    @pl.loop(0, NB, step=2)
    def _batch(bi):
        phase(bi, 0)

        @pl.when(bi + 1 < NB)
        def _(): phase(bi + 1, 1)

    wait_sa(0)
    wait_sd(0)
    wait_sa(1)
    wait_sd(1)
    plsc.subcore_barrier()

    @pl.loop(0, ROWS_PT // B)
    def _flush(j):
        pltpu.sync_copy(acc.at[pl.ds(r0 + j * B, B)], zb0)
        pltpu.sync_copy(zb0, msg_hbm.at[cid, pl.ds(r0 + j * B, B)])

    pltpu.sync_copy(den.at[pl.ds(d0, DEN_PT)], zb0.at[pl.ds(0, DEN_PT)])
    pltpu.sync_copy(zb0.at[pl.ds(0, DEN_PT)], den_hbm.at[cid, pl.ds(d0, DEN_PT)])


def _sc_params():
    cp = pltpu.CompilerParams()
    if "needs_layout_passes" in pltpu.CompilerParams.__dataclass_fields__:
        cp = dataclasses.replace(cp, needs_layout_passes=False)
    return cp


@jax.jit
def _sc_layer(z_flat, elr_flat, srcadj, dstadj):
    mesh = plsc.VectorSubcoreMesh(core_axis_name="c", subcore_axis_name="s")
    logits = functools.partial(
        pl.kernel,
        mesh=mesh,
        compiler_params=_sc_params(),
        out_type=jax.ShapeDtypeStruct((2 * E * 4,), jnp.float32),
        scratch_types=(
            [pltpu.VMEM((N * 8,), jnp.float32)] +
            [pltpu.VMEM((BA,), jnp.int32)] * 4 +
            [pltpu.VMEM((BA * 4,), jnp.float32)] * 2 +
            [pltpu.SemaphoreType.DMA] * 6
        ),
    )(_sc_logits_body)
    w_edges = logits(elr_flat, srcadj, dstadj)

    agg = functools.partial(
        pl.kernel,
        mesh=mesh,
        compiler_params=_sc_params(),
        out_type=[
            jax.ShapeDtypeStruct((2, NPAD, DIM), jnp.float32),
            jax.ShapeDtypeStruct((2, DEN_R, DIM), jnp.float32),
        ],
        scratch_types=(
            [pltpu.VMEM((B,), jnp.int32)] * 8 +
            [pltpu.VMEM((B, DIM), jnp.float32)] * 2 +
            [pltpu.VMEM((B, DIM), jnp.float32)] * 2 +
            [pltpu.VMEM((B * 4,), jnp.float32)] * 2 +
            [pltpu.VMEM_SHARED((NPAD, DIM), jnp.float32),
             pltpu.VMEM_SHARED((DEN_R, DIM), jnp.float32)] +
            [pltpu.SemaphoreType.DMA] * 12
        ),
    )(_sc_agg_body)
    return agg(z_flat, w_edges, srcadj, dstadj)


def _fold(W, a):
    return jnp.sum(W.reshape(DIM, HEADS, D_HEAD) * a[None, :, :], axis=-1)


def _wlr(W, al, ar):
    return jnp.concatenate([_fold(W, al), _fold(W, ar)], axis=1)


def kernel(x, ei0_cites, ei0_writes, ei1_cites, ei1_writes, W0_cites, al0_cites, ar0_cites, W0_writes, al0_writes, ar0_writes, W1_cites, al1_cites, ar1_cites, W1_writes, al1_writes, ar1_writes, W_lin, b_lin):
    w0 = jnp.stack([W0_cites, W0_writes])
    wlr0 = jnp.stack([_wlr(W0_cites, al0_cites, ar0_cites),
                      _wlr(W0_writes, al0_writes, ar0_writes)])
    w1 = jnp.stack([W1_cites, W1_writes])
    wlr1 = jnp.stack([_wlr(W1_cites, al1_cites, ar1_cites),
                      _wlr(W1_writes, al1_writes, ar1_writes)])

    srcadj0 = jnp.concatenate([ei0_cites[0], ei0_writes[0] + N])
    dstadj0 = jnp.concatenate([ei0_cites[1], ei0_writes[1] + N])
    srcadj1 = jnp.concatenate([ei1_cites[0], ei1_writes[0] + N])
    dstadj1 = jnp.concatenate([ei1_cites[1], ei1_writes[1] + N])

    z0, elr0 = _proj(x, w0, wlr0)
    msg0, den0 = _sc_layer(z0.reshape(2 * N, DIM), elr0.reshape(2, N * 8),
                           srcadj0, dstadj0)
    z1, elr1 = _mid(msg0, den0.reshape(2, DEN_R * 32, HEADS), w1, wlr1)
    msg1, den1 = _sc_layer(z1.reshape(2 * N, DIM), elr1.reshape(2, N * 8),
                           srcadj1, dstadj1)
    return _fin(msg1, den1.reshape(2, DEN_R * 32, HEADS), W_lin, b_lin)


# speedup vs baseline: 85.3002x; 1.0737x over previous
import dataclasses
import functools

import jax
import jax.numpy as jnp
from jax import lax
from jax.experimental import pallas as pl
from jax.experimental.pallas import tpu as pltpu
from jax.experimental.pallas import tpu_sc as plsc

N = 10000
E = 160000
DIM = 128
HEADS = 4
D_HEAD = 32
NPAD = 10240
B = 80
NB = (E // 16) // B
ROWS_PT = NPAD // 16
DEN_R = 512
DEN_PT = DEN_R // 16
BLK = 1000
EPS = 1e-9


def _proj_body(x_ref, w_ref, wlr_ref, z_ref, elr_ref):
    xb = x_ref[...]
    z_ref[0] = jnp.dot(xb, w_ref[0], preferred_element_type=jnp.float32)
    elr_ref[0] = jnp.dot(xb, wlr_ref[0], preferred_element_type=jnp.float32)


def _proj(x, w_stack, wlr_stack):
    return pl.pallas_call(
        _proj_body,
        grid=(2, N // BLK),
        in_specs=[
            pl.BlockSpec((BLK, DIM), lambda e, i: (i, 0)),
            pl.BlockSpec((1, DIM, DIM), lambda e, i: (e, 0, 0)),
            pl.BlockSpec((1, DIM, 8), lambda e, i: (e, 0, 0)),
        ],
        out_specs=[
            pl.BlockSpec((1, BLK, DIM), lambda e, i: (e, i, 0)),
            pl.BlockSpec((1, BLK, 8), lambda e, i: (e, i, 0)),
        ],
        out_shape=[
            jax.ShapeDtypeStruct((2, N, DIM), jnp.float32),
            jax.ShapeDtypeStruct((2, N, 8), jnp.float32),
        ],
    )(x, w_stack, wlr_stack)


def _norm(m_ref, d_ref):
    def one(a, d4):
        dexp = jnp.broadcast_to(d4[:, :, None], (BLK, HEADS, D_HEAD))
        dexp = dexp.reshape(BLK, DIM)
        return a / (dexp + EPS)

    return one(m_ref[0], d_ref[0]) + one(m_ref[1], d_ref[1])


def _mid_body(m_ref, d_ref, w_ref, wlr_ref, z_ref, elr_ref):
    h = jnp.maximum(_norm(m_ref, d_ref), 0.0)
    z_ref[0] = jnp.dot(h, w_ref[0], preferred_element_type=jnp.float32)
    elr_ref[0] = jnp.dot(h, wlr_ref[0], preferred_element_type=jnp.float32)


def _mid(msg, den, w_stack, wlr_stack):
    return pl.pallas_call(
        _mid_body,
        grid=(2, N // BLK),
        in_specs=[
            pl.BlockSpec((2, BLK, DIM), lambda e, i: (0, i, 0)),
            pl.BlockSpec((2, BLK, HEADS), lambda e, i: (0, i, 0)),
            pl.BlockSpec((1, DIM, DIM), lambda e, i: (e, 0, 0)),
            pl.BlockSpec((1, DIM, 8), lambda e, i: (e, 0, 0)),
        ],
        out_specs=[
            pl.BlockSpec((1, BLK, DIM), lambda e, i: (e, i, 0)),
            pl.BlockSpec((1, BLK, 8), lambda e, i: (e, i, 0)),
        ],
        out_shape=[
            jax.ShapeDtypeStruct((2, N, DIM), jnp.float32),
            jax.ShapeDtypeStruct((2, N, 8), jnp.float32),
        ],
    )(msg, den, w_stack, wlr_stack)


def _fin_body(m_ref, d_ref, w_ref, b_ref, y_ref):
    h = _norm(m_ref, d_ref)
    y_ref[...] = jnp.dot(h, w_ref[...], preferred_element_type=jnp.float32) + b_ref[...]


def _fin(msg, den, w_lin, b_lin):
    return pl.pallas_call(
        _fin_body,
        grid=(N // BLK,),
        in_specs=[
            pl.BlockSpec((2, BLK, DIM), lambda i: (0, i, 0)),
            pl.BlockSpec((2, BLK, HEADS), lambda i: (0, i, 0)),
            pl.BlockSpec((DIM, DIM), lambda i: (0, 0)),
            pl.BlockSpec((DIM,), lambda i: (0,)),
        ],
        out_specs=pl.BlockSpec((BLK, DIM), lambda i: (i, 0)),
        out_shape=jax.ShapeDtypeStruct((N, DIM), jnp.float32),
    )(msg, den, w_lin, b_lin)


BA = 400
NBA = (E // 16) // BA


def _sc_logits_body(elr_hbm, srcadj_hbm, dstadj_hbm, w_hbm,
                    elr_t, src0, src1, dst0, dst1, wb0, wb1,
                    ssrc0, ssrc1, sdst0, sdst1, sww0, sww1):
    cid = lax.axis_index("c")
    sid = lax.axis_index("s")

    SRC = (src0, src1)
    DST = (dst0, dst1)
    WB = (wb0, wb1)
    SSRC = (ssrc0, ssrc1)
    SDST = (sdst0, sdst1)
    SWW = (sww0, sww1)

    pltpu.sync_copy(elr_hbm.at[cid], elr_t)

    ebase = cid * E + sid * (E // 16)
    nloc8 = cid * (N * 8)

    def issue_idx(bi, p):
        base = ebase + bi * BA
        pltpu.async_copy(srcadj_hbm.at[pl.ds(base, BA)], SRC[p], SSRC[p])
        pltpu.async_copy(dstadj_hbm.at[pl.ds(base, BA)], DST[p], SDST[p])

    def wait_idx(p):
        pltpu.make_async_copy(srcadj_hbm.at[pl.ds(0, BA)], SRC[p], SSRC[p]).wait()
        pltpu.make_async_copy(dstadj_hbm.at[pl.ds(0, BA)], DST[p], SDST[p]).wait()

    def wait_w(p):
        pltpu.make_async_copy(WB[p], w_hbm.at[pl.ds(0, BA * 4)], SWW[p]).wait()

    def phase(bi, p):
        @pl.when(bi >= 2)
        def _(): wait_w(p)

        @pl.loop(0, BA // 16)
        def _grp(g):
            sv = SRC[p][pl.ds(g * 16, 16)] * 8 - nloc8
            dv = DST[p][pl.ds(g * 16, 16)] * 8 - nloc8
            rows = lax.iota(jnp.int32, 16) + g * 16
            for h in range(HEADS):
                elv = plsc.load_gather(elr_t, [sv + h])
                erv = plsc.load_gather(elr_t, [dv + (4 + h)])
                e = elv + erv
                e = jnp.where(e >= 0.0, e, e * jnp.float32(0.2))
                plsc.store_scatter(WB[p], [rows * 4 + h], jnp.exp(e))

        base = ebase + bi * BA
        pltpu.async_copy(WB[p], w_hbm.at[pl.ds(base * 4, BA * 4)], SWW[p])

        q = 1 - p

        @pl.when(bi + 1 < NBA)
        def _():
            wait_idx(q)

            @pl.when(bi + 2 < NBA)
            def _(): issue_idx(bi + 2, p)

    issue_idx(0, 0)
    wait_idx(0)
    issue_idx(1, 1)

    @pl.loop(0, NBA, step=2)
    def _batch(bi):
        phase(bi, 0)

        @pl.when(bi + 1 < NBA)
        def _(): phase(bi + 1, 1)

    wait_w(0)
    wait_w(1)


def _sc_agg_body(z_hbm, w_hbm, srcadj_hbm, dstadj_hbm, msg_hbm, den_hbm,
                 src0, src1, dst0, dst1, dl0, dl1, dd0, dd1, zb0, zb1,
                 ud0, ud1, wb0, wb1, acc, den,
                 ssrc0, ssrc1, sdst0, sdst1, sz0, sz1, sw0, sw1,
                 sa0, sa1, sd0, sd1):
    cid = lax.axis_index("c")
    sid = lax.axis_index("s")

    SRC = (src0, src1)
    DST = (dst0, dst1)
    DL = (dl0, dl1)
    DD = (dd0, dd1)
    ZB = (zb0, zb1)
    UD = (ud0, ud1)
    WB = (wb0, wb1)
    SSRC = (ssrc0, ssrc1)
    SDST = (sdst0, sdst1)
    SZ = (sz0, sz1)
    SW = (sw0, sw1)
    SA = (sa0, sa1)
    SD = (sd0, sd1)

    @pl.loop(0, B)
    def _zero_bufs(r):
        for c in range(DIM // 16):
            zb0[r, pl.ds(c * 16, 16)] = jnp.zeros((16,), jnp.float32)
            ud0[r, pl.ds(c * 16, 16)] = jnp.zeros((16,), jnp.float32)
            ud1[r, pl.ds(c * 16, 16)] = jnp.zeros((16,), jnp.float32)

    for g in range(B // 16):
        dl0[pl.ds(g * 16, 16)] = jnp.zeros((16,), jnp.int32)
        dl1[pl.ds(g * 16, 16)] = jnp.zeros((16,), jnp.int32)

    r0 = sid * ROWS_PT
    d0 = sid * DEN_PT

    @pl.loop(0, ROWS_PT // B)
    def _zero_acc(j):
        pltpu.sync_copy(zb0, acc.at[pl.ds(r0 + j * B, B)])

    pltpu.sync_copy(zb0.at[pl.ds(0, DEN_PT)], den.at[pl.ds(d0, DEN_PT)])
    plsc.subcore_barrier()

    ebase = cid * E + sid * (E // 16)
    nloc = cid * N

    def issue_idx(bi, p):
        base = ebase + bi * B
        pltpu.async_copy(srcadj_hbm.at[pl.ds(base, B)], SRC[p], SSRC[p])
        pltpu.async_copy(dstadj_hbm.at[pl.ds(base, B)], DST[p], SDST[p])

    def wait_idx(p):
        pltpu.make_async_copy(srcadj_hbm.at[pl.ds(0, B)], SRC[p], SSRC[p]).wait()
        pltpu.make_async_copy(dstadj_hbm.at[pl.ds(0, B)], DST[p], SDST[p]).wait()

    def issue_zw(bi, p):
        base = ebase + bi * B
        pltpu.async_copy(z_hbm.at[SRC[p]], ZB[p], SZ[p])
        pltpu.async_copy(w_hbm.at[pl.ds(base * 4, B * 4)], WB[p], SW[p])

    def wait_sa(p):
        pltpu.make_async_copy(ZB[p], acc.at[DL[p]], SA[p]).wait()

    def wait_sd(p):
        pltpu.make_async_copy(UD[p], den.at[DD[p]], SD[p]).wait()

    def phase(bi, p):
        @pl.when(bi >= 2)
        def _(): wait_sd(p)

        for g in range(B // 16):
            dv = DL[p][pl.ds(g * 16, 16)]
            dcol = (dv & 31) * 4
            rows = lax.iota(jnp.int32, 16) + g * 16
            z16 = jnp.zeros((16,), jnp.float32)
            for h in range(HEADS):
                plsc.store_scatter(UD[p], [rows, dcol + h], z16)

        q = 1 - p
        for g in range(B // 16):
            dv = DST[p][pl.ds(g * 16, 16)] - nloc
            DL[p][pl.ds(g * 16, 16)] = dv
            DD[p][pl.ds(g * 16, 16)] = lax.shift_right_logical(dv, 5)

        @pl.when(bi + 1 < NB)
        def _():
            wait_idx(q)

            @pl.when(bi + 1 >= 2)
            def _(): wait_sa(q)

            issue_zw(bi + 1, q)

        pltpu.make_async_copy(w_hbm.at[pl.ds(0, B * 4)], WB[p], SW[p]).wait()
        for g in range(B // 16):
            dv = DL[p][pl.ds(g * 16, 16)]
            dcol = (dv & 31) * 4
            rows = lax.iota(jnp.int32, 16) + g * 16
            for h in range(HEADS):
                w = plsc.load_gather(WB[p], [rows * 4 + h])
                plsc.store_scatter(UD[p], [rows, dcol + h], w)

        pltpu.make_async_copy(z_hbm.at[SRC[p]], ZB[p], SZ[p]).wait()

        @pl.loop(0, B // 16)
        def _scale(g):
            wv = [plsc.load_gather(WB[p], [(lax.iota(jnp.int32, 16) + g * 16) * 4 + h])
                  for h in range(HEADS)]
            for j in range(16):
                for h in range(HEADS):
                    ws = wv[h][j]
                    for k in range(2):
                        off = h * D_HEAD + k * 16
                        zrow = g * 16 + j
                        ZB[p][zrow, pl.ds(off, 16)] = ZB[p][zrow, pl.ds(off, 16)] * ws

        pltpu.async_copy(ZB[p], acc.at[DL[p]], SA[p], add=True)
        pltpu.async_copy(UD[p], den.at[DD[p]], SD[p], add=True)

        @pl.when(bi + 2 < NB)
        def _(): issue_idx(bi + 2, p)

    issue_idx(0, 0)
    wait_idx(0)
    issue_zw(0, 0)
    issue_idx(1, 1)

    @pl.loop(0, NB, step=2)
    def _batch(bi):
        phase(bi, 0)

        @pl.when(bi + 1 < NB)
        def _(): phase(bi + 1, 1)

    wait_sa(0)
    wait_sd(0)
    wait_sa(1)
    wait_sd(1)
    plsc.subcore_barrier()

    @pl.loop(0, ROWS_PT // B)
    def _flush(j):
        pltpu.sync_copy(acc.at[pl.ds(r0 + j * B, B)], zb0)
        pltpu.sync_copy(zb0, msg_hbm.at[cid, pl.ds(r0 + j * B, B)])

    pltpu.sync_copy(den.at[pl.ds(d0, DEN_PT)], zb0.at[pl.ds(0, DEN_PT)])
    pltpu.sync_copy(zb0.at[pl.ds(0, DEN_PT)], den_hbm.at[cid, pl.ds(d0, DEN_PT)])


def _sc_params():
    cp = pltpu.CompilerParams()
    if "needs_layout_passes" in pltpu.CompilerParams.__dataclass_fields__:
        cp = dataclasses.replace(cp, needs_layout_passes=False)
    return cp


@jax.jit
def _sc_layer(z_flat, elr_flat, srcadj, dstadj):
    mesh = plsc.VectorSubcoreMesh(core_axis_name="c", subcore_axis_name="s")
    logits = functools.partial(
        pl.kernel,
        mesh=mesh,
        compiler_params=_sc_params(),
        out_type=jax.ShapeDtypeStruct((2 * E * 4,), jnp.float32),
        scratch_types=(
            [pltpu.VMEM((N * 8,), jnp.float32)] +
            [pltpu.VMEM((BA,), jnp.int32)] * 4 +
            [pltpu.VMEM((BA * 4,), jnp.float32)] * 2 +
            [pltpu.SemaphoreType.DMA] * 6
        ),
    )(_sc_logits_body)
    w_edges = logits(elr_flat, srcadj, dstadj)

    agg = functools.partial(
        pl.kernel,
        mesh=mesh,
        compiler_params=_sc_params(),
        out_type=[
            jax.ShapeDtypeStruct((2, NPAD, DIM), jnp.float32),
            jax.ShapeDtypeStruct((2, DEN_R, DIM), jnp.float32),
        ],
        scratch_types=(
            [pltpu.VMEM((B,), jnp.int32)] * 8 +
            [pltpu.VMEM((B, DIM), jnp.float32)] * 2 +
            [pltpu.VMEM((B, DIM), jnp.float32)] * 2 +
            [pltpu.VMEM((B * 4,), jnp.float32)] * 2 +
            [pltpu.VMEM_SHARED((NPAD, DIM), jnp.float32),
             pltpu.VMEM_SHARED((DEN_R, DIM), jnp.float32)] +
            [pltpu.SemaphoreType.DMA] * 12
        ),
    )(_sc_agg_body)
    return agg(z_flat, w_edges, srcadj, dstadj)


def _fold(W, a):
    return jnp.sum(W.reshape(DIM, HEADS, D_HEAD) * a[None, :, :], axis=-1)


def _wlr(W, al, ar):
    return jnp.concatenate([_fold(W, al), _fold(W, ar)], axis=1)


def kernel(x, ei0_cites, ei0_writes, ei1_cites, ei1_writes, W0_cites, al0_cites, ar0_cites, W0_writes, al0_writes, ar0_writes, W1_cites, al1_cites, ar1_cites, W1_writes, al1_writes, ar1_writes, W_lin, b_lin):
    w0 = jnp.stack([W0_cites, W0_writes])
    wlr0 = jnp.stack([_wlr(W0_cites, al0_cites, ar0_cites),
                      _wlr(W0_writes, al0_writes, ar0_writes)])
    w1 = jnp.stack([W1_cites, W1_writes])
    wlr1 = jnp.stack([_wlr(W1_cites, al1_cites, ar1_cites),
                      _wlr(W1_writes, al1_writes, ar1_writes)])

    srcadj0 = jnp.concatenate([ei0_cites[0], ei0_writes[0] + N])
    dstadj0 = jnp.concatenate([ei0_cites[1], ei0_writes[1] + N])
    srcadj1 = jnp.concatenate([ei1_cites[0], ei1_writes[0] + N])
    dstadj1 = jnp.concatenate([ei1_cites[1], ei1_writes[1] + N])

    z0, elr0 = _proj(x, w0, wlr0)
    msg0, den0 = _sc_layer(z0.reshape(2 * N, DIM), elr0.reshape(2, N * 8),
                           srcadj0, dstadj0)
    z1, elr1 = _mid(msg0, den0.reshape(2, DEN_R * 32, HEADS), w1, wlr1)
    msg1, den1 = _sc_layer(z1.reshape(2 * N, DIM), elr1.reshape(2, N * 8),
                           srcadj1, dstadj1)
    return _fin(msg1, den1.reshape(2, DEN_R * 32, HEADS), W_lin, b_lin)


# speedup vs baseline: 88.9354x; 1.0426x over previous
import dataclasses
import functools

import jax
import jax.numpy as jnp
from jax import lax
from jax.experimental import pallas as pl
from jax.experimental.pallas import tpu as pltpu
from jax.experimental.pallas import tpu_sc as plsc

N = 10000
E = 160000
DIM = 128
HEADS = 4
D_HEAD = 32
NPAD = 10240
B = 80
NB = (E // 16) // B
ROWS_PT = NPAD // 16
DEN_R = 512
DEN_PT = DEN_R // 16
BLK = 1000
EPS = 1e-9


def _proj_body(x_ref, w_ref, wlr_ref, z_ref, elr_ref):
    xb = x_ref[...]
    z_ref[0] = jnp.dot(xb, w_ref[0], preferred_element_type=jnp.float32)
    elr_ref[0] = jnp.dot(xb, wlr_ref[0], preferred_element_type=jnp.float32)


def _proj(x, w_stack, wlr_stack):
    return pl.pallas_call(
        _proj_body,
        grid=(2, N // BLK),
        in_specs=[
            pl.BlockSpec((BLK, DIM), lambda e, i: (i, 0)),
            pl.BlockSpec((1, DIM, DIM), lambda e, i: (e, 0, 0)),
            pl.BlockSpec((1, DIM, 8), lambda e, i: (e, 0, 0)),
        ],
        out_specs=[
            pl.BlockSpec((1, BLK, DIM), lambda e, i: (e, i, 0)),
            pl.BlockSpec((1, BLK, 8), lambda e, i: (e, i, 0)),
        ],
        out_shape=[
            jax.ShapeDtypeStruct((2, N, DIM), jnp.float32),
            jax.ShapeDtypeStruct((2, N, 8), jnp.float32),
        ],
    )(x, w_stack, wlr_stack)


def _norm(m_ref, d_ref):
    def one(a, d4):
        dexp = jnp.broadcast_to(d4[:, :, None], (BLK, HEADS, D_HEAD))
        dexp = dexp.reshape(BLK, DIM)
        return a / (dexp + EPS)

    return one(m_ref[0], d_ref[0]) + one(m_ref[1], d_ref[1])


def _mid_body(m_ref, d_ref, w_ref, wlr_ref, z_ref, elr_ref):
    h = jnp.maximum(_norm(m_ref, d_ref), 0.0)
    z_ref[0] = jnp.dot(h, w_ref[0], preferred_element_type=jnp.float32)
    elr_ref[0] = jnp.dot(h, wlr_ref[0], preferred_element_type=jnp.float32)


def _mid(msg, den, w_stack, wlr_stack):
    return pl.pallas_call(
        _mid_body,
        grid=(2, N // BLK),
        in_specs=[
            pl.BlockSpec((2, BLK, DIM), lambda e, i: (0, i, 0)),
            pl.BlockSpec((2, BLK, HEADS), lambda e, i: (0, i, 0)),
            pl.BlockSpec((1, DIM, DIM), lambda e, i: (e, 0, 0)),
            pl.BlockSpec((1, DIM, 8), lambda e, i: (e, 0, 0)),
        ],
        out_specs=[
            pl.BlockSpec((1, BLK, DIM), lambda e, i: (e, i, 0)),
            pl.BlockSpec((1, BLK, 8), lambda e, i: (e, i, 0)),
        ],
        out_shape=[
            jax.ShapeDtypeStruct((2, N, DIM), jnp.float32),
            jax.ShapeDtypeStruct((2, N, 8), jnp.float32),
        ],
    )(msg, den, w_stack, wlr_stack)


def _fin_body(m_ref, d_ref, w_ref, b_ref, y_ref):
    h = _norm(m_ref, d_ref)
    y_ref[...] = jnp.dot(h, w_ref[...], preferred_element_type=jnp.float32) + b_ref[...]


def _fin(msg, den, w_lin, b_lin):
    return pl.pallas_call(
        _fin_body,
        grid=(N // BLK,),
        in_specs=[
            pl.BlockSpec((2, BLK, DIM), lambda i: (0, i, 0)),
            pl.BlockSpec((2, BLK, HEADS), lambda i: (0, i, 0)),
            pl.BlockSpec((DIM, DIM), lambda i: (0, 0)),
            pl.BlockSpec((DIM,), lambda i: (0,)),
        ],
        out_specs=pl.BlockSpec((BLK, DIM), lambda i: (i, 0)),
        out_shape=jax.ShapeDtypeStruct((N, DIM), jnp.float32),
    )(msg, den, w_lin, b_lin)


BA = 400
NBA = (E // 16) // BA


def _sc_logits_body(elr_hbm, srcadj_hbm, dstadj_hbm, w_hbm,
                    elr_t, src0, src1, dst0, dst1, wb0, wb1,
                    ssrc0, ssrc1, sdst0, sdst1, sww0, sww1):
    cid = lax.axis_index("c")
    sid = lax.axis_index("s")

    SRC = (src0, src1)
    DST = (dst0, dst1)
    WB = (wb0, wb1)
    SSRC = (ssrc0, ssrc1)
    SDST = (sdst0, sdst1)
    SWW = (sww0, sww1)

    pltpu.sync_copy(elr_hbm.at[cid], elr_t)

    ebase = cid * E + sid * (E // 16)
    nloc8 = cid * (N * 8)

    def issue_idx(bi, p):
        base = ebase + bi * BA
        pltpu.async_copy(srcadj_hbm.at[pl.ds(base, BA)], SRC[p], SSRC[p])
        pltpu.async_copy(dstadj_hbm.at[pl.ds(base, BA)], DST[p], SDST[p])

    def wait_idx(p):
        pltpu.make_async_copy(srcadj_hbm.at[pl.ds(0, BA)], SRC[p], SSRC[p]).wait()
        pltpu.make_async_copy(dstadj_hbm.at[pl.ds(0, BA)], DST[p], SDST[p]).wait()

    def wait_w(p):
        pltpu.make_async_copy(WB[p], w_hbm.at[pl.ds(0, BA * 4)], SWW[p]).wait()

    def phase(bi, p):
        @pl.when(bi >= 2)
        def _(): wait_w(p)

        @pl.loop(0, BA // 16)
        def _grp(g):
            sv = SRC[p][pl.ds(g * 16, 16)] * 8 - nloc8
            dv = DST[p][pl.ds(g * 16, 16)] * 8 - nloc8
            rows = lax.iota(jnp.int32, 16) + g * 16
            for h in range(HEADS):
                elv = plsc.load_gather(elr_t, [sv + h])
                erv = plsc.load_gather(elr_t, [dv + (4 + h)])
                e = elv + erv
                e = jnp.where(e >= 0.0, e, e * jnp.float32(0.2))
                plsc.store_scatter(WB[p], [rows * 4 + h], jnp.exp(e))

        base = ebase + bi * BA
        pltpu.async_copy(WB[p], w_hbm.at[pl.ds(base * 4, BA * 4)], SWW[p])

        q = 1 - p

        @pl.when(bi + 1 < NBA)
        def _():
            wait_idx(q)

            @pl.when(bi + 2 < NBA)
            def _(): issue_idx(bi + 2, p)

    issue_idx(0, 0)
    wait_idx(0)
    issue_idx(1, 1)

    @pl.loop(0, NBA, step=2)
    def _batch(bi):
        phase(bi, 0)

        @pl.when(bi + 1 < NBA)
        def _(): phase(bi + 1, 1)

    wait_w(0)
    wait_w(1)


def _sc_agg_body(z_hbm, w_hbm, srcadj_hbm, dstadj_hbm, msg_hbm, den_hbm,
                 src0, src1, dst0, dst1, dl0, dl1, dd0, dd1, zb0, zb1,
                 ud0, ud1, wb0, wb1, acc, den,
                 ssrc0, ssrc1, sdst0, sdst1, sz0, sz1, sw0, sw1,
                 sa0, sa1, sd0, sd1):
    cid = lax.axis_index("c")
    sid = lax.axis_index("s")

    SRC = (src0, src1)
    DST = (dst0, dst1)
    DL = (dl0, dl1)
    DD = (dd0, dd1)
    ZB = (zb0, zb1)
    UD = (ud0, ud1)
    WB = (wb0, wb1)
    SSRC = (ssrc0, ssrc1)
    SDST = (sdst0, sdst1)
    SZ = (sz0, sz1)
    SW = (sw0, sw1)
    SA = (sa0, sa1)
    SD = (sd0, sd1)

    @pl.loop(0, B)
    def _zero_bufs(r):
        for c in range(DIM // 16):
            zb0[r, pl.ds(c * 16, 16)] = jnp.zeros((16,), jnp.float32)
            ud0[r, pl.ds(c * 16, 16)] = jnp.zeros((16,), jnp.float32)
            ud1[r, pl.ds(c * 16, 16)] = jnp.zeros((16,), jnp.float32)

    for g in range(B // 16):
        dl0[pl.ds(g * 16, 16)] = jnp.zeros((16,), jnp.int32)
        dl1[pl.ds(g * 16, 16)] = jnp.zeros((16,), jnp.int32)

    r0 = sid * ROWS_PT
    d0 = sid * DEN_PT

    @pl.loop(0, ROWS_PT // B)
    def _zero_acc(j):
        pltpu.sync_copy(zb0, acc.at[pl.ds(r0 + j * B, B)])

    pltpu.sync_copy(zb0.at[pl.ds(0, DEN_PT)], den.at[pl.ds(d0, DEN_PT)])
    plsc.subcore_barrier()

    ebase = cid * E + sid * (E // 16)
    nloc = cid * N

    def issue_idx(bi, p):
        base = ebase + bi * B
        pltpu.async_copy(srcadj_hbm.at[pl.ds(base, B)], SRC[p], SSRC[p])
        pltpu.async_copy(dstadj_hbm.at[pl.ds(base, B)], DST[p], SDST[p])

    def wait_idx(p):
        pltpu.make_async_copy(srcadj_hbm.at[pl.ds(0, B)], SRC[p], SSRC[p]).wait()
        pltpu.make_async_copy(dstadj_hbm.at[pl.ds(0, B)], DST[p], SDST[p]).wait()

    def issue_zw(bi, p):
        base = ebase + bi * B
        pltpu.async_copy(z_hbm.at[SRC[p]], ZB[p], SZ[p])
        pltpu.async_copy(w_hbm.at[pl.ds(base * 4, B * 4)], WB[p], SW[p])

    def wait_sa(p):
        pltpu.make_async_copy(ZB[p], acc.at[DL[p]], SA[p]).wait()

    def wait_sd(p):
        pltpu.make_async_copy(UD[p], den.at[DD[p]], SD[p]).wait()

    def phase(bi, p):
        @pl.when(bi >= 2)
        def _(): wait_sd(p)

        for g in range(B // 16):
            dv = DL[p][pl.ds(g * 16, 16)]
            dcol = (dv & 31) * 4
            rows = lax.iota(jnp.int32, 16) + g * 16
            z16 = jnp.zeros((16,), jnp.float32)
            for h in range(HEADS):
                plsc.store_scatter(UD[p], [rows, dcol + h], z16)

        q = 1 - p
        for g in range(B // 16):
            dv = DST[p][pl.ds(g * 16, 16)] - nloc
            DL[p][pl.ds(g * 16, 16)] = dv
            DD[p][pl.ds(g * 16, 16)] = lax.shift_right_logical(dv, 5)

        @pl.when(bi + 1 < NB)
        def _():
            wait_idx(q)

            @pl.when(bi + 1 >= 2)
            def _(): wait_sa(q)

            issue_zw(bi + 1, q)

        pltpu.make_async_copy(w_hbm.at[pl.ds(0, B * 4)], WB[p], SW[p]).wait()
        pltpu.make_async_copy(z_hbm.at[SRC[p]], ZB[p], SZ[p]).wait()

        @pl.loop(0, B // 16)
        def _scale(g):
            rows = lax.iota(jnp.int32, 16) + g * 16
            wv = [plsc.load_gather(WB[p], [rows * 4 + h]) for h in range(HEADS)]
            dv = DL[p][pl.ds(g * 16, 16)]
            dcol = (dv & 31) * 4
            for h in range(HEADS):
                plsc.store_scatter(UD[p], [rows, dcol + h], wv[h])
            for j in range(16):
                for h in range(HEADS):
                    ws = wv[h][j]
                    for k in range(2):
                        off = h * D_HEAD + k * 16
                        zrow = g * 16 + j
                        ZB[p][zrow, pl.ds(off, 16)] = ZB[p][zrow, pl.ds(off, 16)] * ws

        pltpu.async_copy(ZB[p], acc.at[DL[p]], SA[p], add=True)
        pltpu.async_copy(UD[p], den.at[DD[p]], SD[p], add=True)

        @pl.when(bi + 2 < NB)
        def _(): issue_idx(bi + 2, p)

    issue_idx(0, 0)
    wait_idx(0)
    issue_zw(0, 0)
    issue_idx(1, 1)

    @pl.loop(0, NB, step=2)
    def _batch(bi):
        phase(bi, 0)

        @pl.when(bi + 1 < NB)
        def _(): phase(bi + 1, 1)

    wait_sa(0)
    wait_sd(0)
    wait_sa(1)
    wait_sd(1)
    plsc.subcore_barrier()

    @pl.loop(0, ROWS_PT // B)
    def _flush(j):
        pltpu.sync_copy(acc.at[pl.ds(r0 + j * B, B)], zb0)
        pltpu.sync_copy(zb0, msg_hbm.at[cid, pl.ds(r0 + j * B, B)])

    pltpu.sync_copy(den.at[pl.ds(d0, DEN_PT)], zb0.at[pl.ds(0, DEN_PT)])
    pltpu.sync_copy(zb0.at[pl.ds(0, DEN_PT)], den_hbm.at[cid, pl.ds(d0, DEN_PT)])


def _sc_params():
    cp = pltpu.CompilerParams()
    if "needs_layout_passes" in pltpu.CompilerParams.__dataclass_fields__:
        cp = dataclasses.replace(cp, needs_layout_passes=False)
    return cp


@jax.jit
def _sc_layer(z_flat, elr_flat, srcadj, dstadj):
    mesh = plsc.VectorSubcoreMesh(core_axis_name="c", subcore_axis_name="s")
    logits = functools.partial(
        pl.kernel,
        mesh=mesh,
        compiler_params=_sc_params(),
        out_type=jax.ShapeDtypeStruct((2 * E * 4,), jnp.float32),
        scratch_types=(
            [pltpu.VMEM((N * 8,), jnp.float32)] +
            [pltpu.VMEM((BA,), jnp.int32)] * 4 +
            [pltpu.VMEM((BA * 4,), jnp.float32)] * 2 +
            [pltpu.SemaphoreType.DMA] * 6
        ),
    )(_sc_logits_body)
    w_edges = logits(elr_flat, srcadj, dstadj)

    agg = functools.partial(
        pl.kernel,
        mesh=mesh,
        compiler_params=_sc_params(),
        out_type=[
            jax.ShapeDtypeStruct((2, NPAD, DIM), jnp.float32),
            jax.ShapeDtypeStruct((2, DEN_R, DIM), jnp.float32),
        ],
        scratch_types=(
            [pltpu.VMEM((B,), jnp.int32)] * 8 +
            [pltpu.VMEM((B, DIM), jnp.float32)] * 2 +
            [pltpu.VMEM((B, DIM), jnp.float32)] * 2 +
            [pltpu.VMEM((B * 4,), jnp.float32)] * 2 +
            [pltpu.VMEM_SHARED((NPAD, DIM), jnp.float32),
             pltpu.VMEM_SHARED((DEN_R, DIM), jnp.float32)] +
            [pltpu.SemaphoreType.DMA] * 12
        ),
    )(_sc_agg_body)
    return agg(z_flat, w_edges, srcadj, dstadj)


def _fold(W, a):
    return jnp.sum(W.reshape(DIM, HEADS, D_HEAD) * a[None, :, :], axis=-1)


def _wlr(W, al, ar):
    return jnp.concatenate([_fold(W, al), _fold(W, ar)], axis=1)


def kernel(x, ei0_cites, ei0_writes, ei1_cites, ei1_writes, W0_cites, al0_cites, ar0_cites, W0_writes, al0_writes, ar0_writes, W1_cites, al1_cites, ar1_cites, W1_writes, al1_writes, ar1_writes, W_lin, b_lin):
    w0 = jnp.stack([W0_cites, W0_writes])
    wlr0 = jnp.stack([_wlr(W0_cites, al0_cites, ar0_cites),
                      _wlr(W0_writes, al0_writes, ar0_writes)])
    w1 = jnp.stack([W1_cites, W1_writes])
    wlr1 = jnp.stack([_wlr(W1_cites, al1_cites, ar1_cites),
                      _wlr(W1_writes, al1_writes, ar1_writes)])

    srcadj0 = jnp.concatenate([ei0_cites[0], ei0_writes[0] + N])
    dstadj0 = jnp.concatenate([ei0_cites[1], ei0_writes[1] + N])
    srcadj1 = jnp.concatenate([ei1_cites[0], ei1_writes[0] + N])
    dstadj1 = jnp.concatenate([ei1_cites[1], ei1_writes[1] + N])

    z0, elr0 = _proj(x, w0, wlr0)
    msg0, den0 = _sc_layer(z0.reshape(2 * N, DIM), elr0.reshape(2, N * 8),
                           srcadj0, dstadj0)
    z1, elr1 = _mid(msg0, den0.reshape(2, DEN_R * 32, HEADS), w1, wlr1)
    msg1, den1 = _sc_layer(z1.reshape(2 * N, DIM), elr1.reshape(2, N * 8),
                           srcadj1, dstadj1)
    return _fin(msg1, den1.reshape(2, DEN_R * 32, HEADS), W_lin, b_lin)
